# baseline (speedup 1.0000x reference)
import jax
import jax.numpy as jnp
from jax import lax
from jax.experimental import pallas as pl
from jax.experimental.pallas import tpu as pltpu
from jax.experimental.pallas import tpu_sc as plsc
import functools

BATCH = 16384
HIST = 50
EMBED_DIM = 32

NC = 2
NS = 16
NW = NC * NS

WORD_NUM = 1000000
WORDS = WORD_NUM * EMBED_DIM

LANES = 128
NBLOCKS = HIST * (BATCH // LANES)
BLOCKS_PER_W = NBLOCKS // NW
TC_PER_H = BATCH // LANES
ETILES = EMBED_DIM // 8


VFULL = 999936
VCHUNK = 512
NCH = VFULL // VCHUNK


def _make_linearizer():
  mesh = plsc.VectorSubcoreMesh(
      core_axis_name="c", subcore_axis_name="s",
      num_cores=NC, num_subcores=NS)

  @functools.partial(
      pl.kernel,
      out_type=jax.ShapeDtypeStruct((WORDS,), jnp.float32),
      mesh=mesh,
      scratch_types=[
          pltpu.VMEM((128, 128), jnp.float32),
          pltpu.VMEM((VCHUNK * EMBED_DIM,), jnp.float32),
          pltpu.VMEM(((WORD_NUM - VFULL) * EMBED_DIM,), jnp.float32),
          pltpu.SemaphoreType.DMA,
          pltpu.SemaphoreType.DMA,
      ],
      compiler_params=pltpu.CompilerParams(use_tc_tiling_on_sc=True,
                                           needs_layout_passes=False),
  )
  def sc_lin(tab_t, tail_hbm, out_hbm, inb, outb, tailv, isem, osem):
    wid = lax.axis_index("s") * NC + lax.axis_index("c")
    lane32 = lax.iota(jnp.int32, 16) * EMBED_DIM

    nch = 61 + (wid == 0).astype(jnp.int32)

    @pl.loop(0, nch)
    def _(k):
      c = wid + k * NW
      v0 = c * VCHUNK
      for tr in range(4):
        for tcq in range(4):
          pltpu.async_copy(
              tab_t.at[pl.ds(tr * 8, 8), pl.ds(v0 + tcq * 128, 128)],
              inb.at[pl.ds((tr * 4 + tcq) * 8, 8)], isem)
      for tr in range(4):
        for tcq in range(4):
          pltpu.make_async_copy(
              tab_t.at[pl.ds(tr * 8, 8), pl.ds(v0 + tcq * 128, 128)],
              inb.at[pl.ds((tr * 4 + tcq) * 8, 8)], isem).wait()

      @plsc.parallel_loop(0, 128, unroll=2)
      def _(r):
        e = (r // 32) * 8 + (r % 8)
        sbase = ((r // 8) % 4) * 4096 + e
        for g in range(8):
          v16 = inb[r, pl.ds(g * 16, 16)]
          plsc.store_scatter(outb, [lane32 + (sbase + g * 512)], v16)

      pltpu.async_copy(outb, out_hbm.at[pl.ds(v0 * EMBED_DIM, VCHUNK * EMBED_DIM)],
                       osem)
      pltpu.make_async_copy(outb,
                            out_hbm.at[pl.ds(v0 * EMBED_DIM, VCHUNK * EMBED_DIM)],
                            osem).wait()

    @pl.when(wid == 1)
    def _():
      pltpu.sync_copy(tail_hbm, tailv)
      pltpu.sync_copy(tailv,
                      out_hbm.at[pl.ds(VFULL * EMBED_DIM,
                                       (WORD_NUM - VFULL) * EMBED_DIM)])

  return sc_lin


def _make_sc_kernel():
  mesh = plsc.VectorSubcoreMesh(
      core_axis_name="c", subcore_axis_name="s",
      num_cores=NC, num_subcores=NS)

  @functools.partial(
      pl.kernel,
      out_type=jax.ShapeDtypeStruct((HIST, ETILES, TC_PER_H, 8, LANES),
                                    jnp.float32),
      mesh=mesh,
      scratch_types=[
          pltpu.VMEM((BLOCKS_PER_W, LANES), jnp.int32),
          pltpu.VMEM((2, LANES, EMBED_DIM), jnp.float32),
          pltpu.VMEM((2, ETILES, 8, LANES), jnp.float32),
          pltpu.SemaphoreType.DMA,
          pltpu.SemaphoreType.DMA,
          pltpu.SemaphoreType.DMA,
          pltpu.SemaphoreType.DMA,
      ],
      compiler_params=pltpu.CompilerParams(use_tc_tiling_on_sc=False,
                                           needs_layout_passes=False),
  )
  def sc_embed(idx_hbm, table_hbm, out_hbm, idx_v, rows_v, trv, g0, g1, s0, s1):
    gsem = [g0, g1]
    ssem = [s0, s1]
    wid = lax.axis_index("s") * NC + lax.axis_index("c")
    base = wid * BLOCKS_PER_W

    pltpu.sync_copy(idx_hbm.at[pl.ds(base, BLOCKS_PER_W)], idx_v)

    lane = lax.iota(jnp.int32, 16)

    def fire_gather(g, b):
      pltpu.async_copy(table_hbm.at[idx_v.at[g]], rows_v.at[b], gsem[b])

    def wait_gather(b):
      pltpu.make_async_copy(table_hbm.at[idx_v.at[0]], rows_v.at[b],
                            gsem[b]).wait()

    def transpose(b):
      @plsc.parallel_loop(0, 8, unroll=2)
      def _(blk):
        rid = lane + blk * 16
        for e in range(EMBED_DIM):
          v = plsc.load_gather(
              rows_v.at[b], [rid, jnp.full((16,), e, jnp.int32)])
          trv[b, e // 8, e % 8, pl.ds(blk * 16, 16)] = v

    def fire_stores(g, b):
      h = g // TC_PER_H
      tc = g % TC_PER_H
      for tr in range(ETILES):
        pltpu.async_copy(trv.at[b, tr], out_hbm.at[h, tr, tc], ssem[b])

    def wait_stores(g, b):
      h = g // TC_PER_H
      tc = g % TC_PER_H
      for tr in range(ETILES):
        pltpu.make_async_copy(trv.at[b, tr], out_hbm.at[h, tr, tc],
                              ssem[b]).wait()

    fire_gather(0, 0)
    fire_gather(1, 1)

    for bsel in range(2):
      wait_gather(bsel)
      transpose(bsel)
      fire_gather(2 + bsel, bsel)
      fire_stores(base + bsel, bsel)

    @pl.loop(1, BLOCKS_PER_W // 2 - 1)
    def _(i):
      for bsel in range(2):
        gl = i * 2 + bsel
        g = base + gl
        wait_gather(bsel)
        wait_stores(g - 2, bsel)
        transpose(bsel)
        fire_gather(gl + 2, bsel)
        fire_stores(g, bsel)

    for bsel in range(2):
      g = base + BLOCKS_PER_W - 2 + bsel
      wait_gather(bsel)
      wait_stores(g - 2, bsel)
      transpose(bsel)
      fire_stores(g, bsel)

    wait_stores(base + BLOCKS_PER_W - 2, 0)
    wait_stores(base + BLOCKS_PER_W - 1, 1)

  return sc_embed


def kernel(x, table):
  idx = x.T.reshape(NBLOCKS, LANES).astype(jnp.int32)
  tail = lax.slice(table, (VFULL, 0), (WORD_NUM, EMBED_DIM)).reshape(-1)
  lin = _make_linearizer()(table.T, tail).reshape(WORD_NUM, EMBED_DIM)
  out5 = _make_sc_kernel()(idx, lin)
  return out5.transpose(2, 4, 0, 1, 3).reshape(BATCH, HIST, EMBED_DIM)


# speedup vs baseline: 1.1402x; 1.1402x over previous
import jax
import jax.numpy as jnp
from jax import lax
from jax.experimental import pallas as pl
from jax.experimental.pallas import tpu as pltpu
from jax.experimental.pallas import tpu_sc as plsc
import functools

BATCH = 16384
HIST = 50
EMBED_DIM = 32

NC = 2
NS = 16
NW = NC * NS

WORD_NUM = 1000000
WORDS = WORD_NUM * EMBED_DIM

LANES = 128
NBLOCKS = HIST * (BATCH // LANES)
BLOCKS_PER_W = NBLOCKS // NW
TC_PER_H = BATCH // LANES
ETILES = EMBED_DIM // 8


VFULL = 999936
VCHUNK = 512
NCH = VFULL // VCHUNK


def _make_linearizer():
  mesh = plsc.VectorSubcoreMesh(
      core_axis_name="c", subcore_axis_name="s",
      num_cores=NC, num_subcores=NS)

  @functools.partial(
      pl.kernel,
      out_type=jax.ShapeDtypeStruct((WORDS,), jnp.float32),
      mesh=mesh,
      scratch_types=[
          pltpu.VMEM((128, 128), jnp.float32),
          pltpu.VMEM((128, 128), jnp.float32),
          pltpu.VMEM((VCHUNK * EMBED_DIM,), jnp.float32),
          pltpu.VMEM((VCHUNK * EMBED_DIM,), jnp.float32),
          pltpu.VMEM(((WORD_NUM - VFULL) * EMBED_DIM,), jnp.float32),
          pltpu.SemaphoreType.DMA,
          pltpu.SemaphoreType.DMA,
          pltpu.SemaphoreType.DMA,
          pltpu.SemaphoreType.DMA,
      ],
      compiler_params=pltpu.CompilerParams(use_tc_tiling_on_sc=True,
                                           needs_layout_passes=False),
  )
  def sc_lin(tab_t, tail_hbm, out_hbm, inb0, inb1, outb0, outb1, tailv,
             i0, i1, o0, o1):
    inb = [inb0, inb1]
    outb = [outb0, outb1]
    isem = [i0, i1]
    osem = [o0, o1]
    wid = lax.axis_index("s") * NC + lax.axis_index("c")
    lane32 = lax.iota(jnp.int32, 16) * EMBED_DIM

    def fire_in(k, b):
      v0 = (wid + k * NW) * VCHUNK
      for tr in range(4):
        for tcq in range(4):
          pltpu.async_copy(
              tab_t.at[pl.ds(tr * 8, 8), pl.ds(v0 + tcq * 128, 128)],
              inb[b].at[pl.ds((tr * 4 + tcq) * 8, 8)], isem[b])

    def wait_in(k, b):
      v0 = (wid + k * NW) * VCHUNK
      for tr in range(4):
        for tcq in range(4):
          pltpu.make_async_copy(
              tab_t.at[pl.ds(tr * 8, 8), pl.ds(v0 + tcq * 128, 128)],
              inb[b].at[pl.ds((tr * 4 + tcq) * 8, 8)], isem[b]).wait()

    def transpose(b):
      @plsc.parallel_loop(0, 128, unroll=2)
      def _(r):
        e = (r // 32) * 8 + (r % 8)
        sbase = ((r // 8) % 4) * 4096 + e
        for g in range(8):
          v16 = inb[b][r, pl.ds(g * 16, 16)]
          plsc.store_scatter(outb[b], [lane32 + (sbase + g * 512)], v16)

    def fire_out(k, b):
      v0 = (wid + k * NW) * VCHUNK
      pltpu.async_copy(
          outb[b],
          out_hbm.at[pl.ds(v0 * EMBED_DIM, VCHUNK * EMBED_DIM)], osem[b])

    def wait_out(k, b):
      v0 = (wid + k * NW) * VCHUNK
      pltpu.make_async_copy(
          outb[b],
          out_hbm.at[pl.ds(v0 * EMBED_DIM, VCHUNK * EMBED_DIM)], osem[b]).wait()

    fire_in(0, 0)

    for b in range(2):
      wait_in(b, b)
      fire_in(b + 1, 1 - b)
      transpose(b)
      fire_out(b, b)

    @pl.loop(1, 30)
    def _(p):
      for b in range(2):
        k = p * 2 + b
        wait_in(k, b)
        fire_in(k + 1, 1 - b)
        wait_out(k - 2, b)
        transpose(b)
        fire_out(k, b)

    wait_in(60, 0)
    wait_out(58, 0)
    transpose(0)
    fire_out(60, 0)

    wait_out(59, 1)

    @pl.when(wid == 0)
    def _():
      fire_in(61, 1)
      wait_in(61, 1)
      transpose(1)
      fire_out(61, 1)
      wait_out(61, 1)

    wait_out(60, 0)

    @pl.when(wid == 1)
    def _():
      pltpu.sync_copy(tail_hbm, tailv)
      pltpu.sync_copy(tailv,
                      out_hbm.at[pl.ds(VFULL * EMBED_DIM,
                                       (WORD_NUM - VFULL) * EMBED_DIM)])

  return sc_lin


def _make_sc_kernel():
  mesh = plsc.VectorSubcoreMesh(
      core_axis_name="c", subcore_axis_name="s",
      num_cores=NC, num_subcores=NS)

  @functools.partial(
      pl.kernel,
      out_type=jax.ShapeDtypeStruct((HIST, ETILES, TC_PER_H, 8, LANES),
                                    jnp.float32),
      mesh=mesh,
      scratch_types=[
          pltpu.VMEM((BLOCKS_PER_W, LANES), jnp.int32),
          pltpu.VMEM((2, LANES, EMBED_DIM), jnp.float32),
          pltpu.VMEM((2, ETILES, 8, LANES), jnp.float32),
          pltpu.SemaphoreType.DMA,
          pltpu.SemaphoreType.DMA,
          pltpu.SemaphoreType.DMA,
          pltpu.SemaphoreType.DMA,
      ],
      compiler_params=pltpu.CompilerParams(use_tc_tiling_on_sc=False,
                                           needs_layout_passes=False),
  )
  def sc_embed(idx_hbm, table_hbm, out_hbm, idx_v, rows_v, trv, g0, g1, s0, s1):
    gsem = [g0, g1]
    ssem = [s0, s1]
    wid = lax.axis_index("s") * NC + lax.axis_index("c")
    base = wid * BLOCKS_PER_W

    pltpu.sync_copy(idx_hbm.at[pl.ds(base, BLOCKS_PER_W)], idx_v)

    lane = lax.iota(jnp.int32, 16)

    def fire_gather(g, b):
      pltpu.async_copy(table_hbm.at[idx_v.at[g]], rows_v.at[b], gsem[b])

    def wait_gather(b):
      pltpu.make_async_copy(table_hbm.at[idx_v.at[0]], rows_v.at[b],
                            gsem[b]).wait()

    def transpose(b):
      @plsc.parallel_loop(0, 8, unroll=2)
      def _(blk):
        rid = lane + blk * 16
        for e in range(EMBED_DIM):
          v = plsc.load_gather(
              rows_v.at[b], [rid, jnp.full((16,), e, jnp.int32)])
          trv[b, e // 8, e % 8, pl.ds(blk * 16, 16)] = v

    def fire_stores(g, b):
      h = g // TC_PER_H
      tc = g % TC_PER_H
      for tr in range(ETILES):
        pltpu.async_copy(trv.at[b, tr], out_hbm.at[h, tr, tc], ssem[b])

    def wait_stores(g, b):
      h = g // TC_PER_H
      tc = g % TC_PER_H
      for tr in range(ETILES):
        pltpu.make_async_copy(trv.at[b, tr], out_hbm.at[h, tr, tc],
                              ssem[b]).wait()

    fire_gather(0, 0)
    fire_gather(1, 1)

    for bsel in range(2):
      wait_gather(bsel)
      transpose(bsel)
      fire_gather(2 + bsel, bsel)
      fire_stores(base + bsel, bsel)

    @pl.loop(1, BLOCKS_PER_W // 2 - 1)
    def _(i):
      for bsel in range(2):
        gl = i * 2 + bsel
        g = base + gl
        wait_gather(bsel)
        wait_stores(g - 2, bsel)
        transpose(bsel)
        fire_gather(gl + 2, bsel)
        fire_stores(g, bsel)

    for bsel in range(2):
      g = base + BLOCKS_PER_W - 2 + bsel
      wait_gather(bsel)
      wait_stores(g - 2, bsel)
      transpose(bsel)
      fire_stores(g, bsel)

    wait_stores(base + BLOCKS_PER_W - 2, 0)
    wait_stores(base + BLOCKS_PER_W - 1, 1)

  return sc_embed


def kernel(x, table):
  idx = x.T.reshape(NBLOCKS, LANES).astype(jnp.int32)
  tail = lax.slice(table, (VFULL, 0), (WORD_NUM, EMBED_DIM)).reshape(-1)
  lin = _make_linearizer()(table.T, tail).reshape(WORD_NUM, EMBED_DIM)
  out5 = _make_sc_kernel()(idx, lin)
  return out5.transpose(2, 4, 0, 1, 3).reshape(BATCH, HIST, EMBED_DIM)


# speedup vs baseline: 1.6892x; 1.4815x over previous
import jax
import jax.numpy as jnp
from jax import lax
from jax.experimental import pallas as pl
from jax.experimental.pallas import tpu as pltpu
from jax.experimental.pallas import tpu_sc as plsc
import functools

BATCH = 16384
HIST = 50
EMBED_DIM = 32

NC = 2
NS = 16
NW = NC * NS

WORD_NUM = 1000000
WORDS = WORD_NUM * EMBED_DIM

LANES = 128
NBLOCKS = HIST * (BATCH // LANES)
BLOCKS_PER_W = NBLOCKS // NW
TC_PER_H = BATCH // LANES
ETILES = EMBED_DIM // 8


VFULL = 999936
VCHUNK = 512
NCH = VFULL // VCHUNK


def _make_linearizer():
  mesh = plsc.VectorSubcoreMesh(
      core_axis_name="c", subcore_axis_name="s",
      num_cores=NC, num_subcores=NS)

  @functools.partial(
      pl.kernel,
      out_type=jax.ShapeDtypeStruct((WORDS,), jnp.float32),
      mesh=mesh,
      scratch_types=[
          pltpu.VMEM((128, 128), jnp.float32),
          pltpu.VMEM((128, 128), jnp.float32),
          pltpu.VMEM((VCHUNK * EMBED_DIM,), jnp.float32),
          pltpu.VMEM((VCHUNK * EMBED_DIM,), jnp.float32),
          pltpu.VMEM(((WORD_NUM - VFULL) * EMBED_DIM,), jnp.float32),
          pltpu.SemaphoreType.DMA,
          pltpu.SemaphoreType.DMA,
          pltpu.SemaphoreType.DMA,
          pltpu.SemaphoreType.DMA,
      ],
      compiler_params=pltpu.CompilerParams(use_tc_tiling_on_sc=True,
                                           needs_layout_passes=False,
                                           disable_bounds_checks=True),
  )
  def sc_lin(tab_t, tail_hbm, out_hbm, inb0, inb1, outb0, outb1, tailv,
             i0, i1, o0, o1):
    inb = [inb0, inb1]
    outb = [outb0, outb1]
    isem = [i0, i1]
    osem = [o0, o1]
    wid = lax.axis_index("s") * NC + lax.axis_index("c")
    lane32 = lax.iota(jnp.int32, 16) * EMBED_DIM

    def fire_in(k, b):
      v0 = (wid + k * NW) * VCHUNK
      for tr in range(4):
        for tcq in range(4):
          pltpu.async_copy(
              tab_t.at[pl.ds(tr * 8, 8), pl.ds(v0 + tcq * 128, 128)],
              inb[b].at[pl.ds((tr * 4 + tcq) * 8, 8)], isem[b])

    def wait_in(k, b):
      v0 = (wid + k * NW) * VCHUNK
      for tr in range(4):
        for tcq in range(4):
          pltpu.make_async_copy(
              tab_t.at[pl.ds(tr * 8, 8), pl.ds(v0 + tcq * 128, 128)],
              inb[b].at[pl.ds((tr * 4 + tcq) * 8, 8)], isem[b]).wait()

    def transpose(b):
      @plsc.parallel_loop(0, 128, unroll=2)
      def _(r):
        e = (r // 32) * 8 + (r % 8)
        sbase = ((r // 8) % 4) * 4096 + e
        for g in range(8):
          v16 = inb[b][r, pl.ds(g * 16, 16)]
          plsc.store_scatter(outb[b], [lane32 + (sbase + g * 512)], v16)

    def fire_out(k, b):
      v0 = (wid + k * NW) * VCHUNK
      pltpu.async_copy(
          outb[b],
          out_hbm.at[pl.ds(v0 * EMBED_DIM, VCHUNK * EMBED_DIM)], osem[b])

    def wait_out(k, b):
      v0 = (wid + k * NW) * VCHUNK
      pltpu.make_async_copy(
          outb[b],
          out_hbm.at[pl.ds(v0 * EMBED_DIM, VCHUNK * EMBED_DIM)], osem[b]).wait()

    fire_in(0, 0)

    for b in range(2):
      wait_in(b, b)
      fire_in(b + 1, 1 - b)
      transpose(b)
      fire_out(b, b)

    @pl.loop(1, 30)
    def _(p):
      for b in range(2):
        k = p * 2 + b
        wait_in(k, b)
        fire_in(k + 1, 1 - b)
        wait_out(k - 2, b)
        transpose(b)
        fire_out(k, b)

    wait_in(60, 0)
    wait_out(58, 0)
    transpose(0)
    fire_out(60, 0)

    wait_out(59, 1)

    @pl.when(wid == 0)
    def _():
      fire_in(61, 1)
      wait_in(61, 1)
      transpose(1)
      fire_out(61, 1)
      wait_out(61, 1)

    wait_out(60, 0)

    @pl.when(wid == 1)
    def _():
      pltpu.sync_copy(tail_hbm, tailv)
      pltpu.sync_copy(tailv,
                      out_hbm.at[pl.ds(VFULL * EMBED_DIM,
                                       (WORD_NUM - VFULL) * EMBED_DIM)])

  return sc_lin


def _make_sc_kernel():
  mesh = plsc.VectorSubcoreMesh(
      core_axis_name="c", subcore_axis_name="s",
      num_cores=NC, num_subcores=NS)

  @functools.partial(
      pl.kernel,
      out_type=jax.ShapeDtypeStruct((HIST, ETILES, TC_PER_H, 8, LANES),
                                    jnp.float32),
      mesh=mesh,
      scratch_types=[
          pltpu.VMEM((BLOCKS_PER_W, LANES), jnp.int32),
          pltpu.VMEM((2, LANES, EMBED_DIM), jnp.float32),
          pltpu.VMEM((2, EMBED_DIM, 129), jnp.float32),
          pltpu.SemaphoreType.DMA,
          pltpu.SemaphoreType.DMA,
          pltpu.SemaphoreType.DMA,
          pltpu.SemaphoreType.DMA,
      ],
      compiler_params=pltpu.CompilerParams(use_tc_tiling_on_sc=False,
                                           needs_layout_passes=False,
                                           disable_bounds_checks=True),
  )
  def sc_embed(idx_hbm, table_hbm, out_hbm, idx_v, rows_v, trv, g0, g1, s0, s1):
    gsem = [g0, g1]
    ssem = [s0, s1]
    wid = lax.axis_index("s") * NC + lax.axis_index("c")
    base = wid * BLOCKS_PER_W

    pltpu.sync_copy(idx_hbm.at[pl.ds(base, BLOCKS_PER_W)], idx_v)

    lane = lax.iota(jnp.int32, 16)

    def fire_gather(g, b):
      pltpu.async_copy(table_hbm.at[idx_v.at[g]], rows_v.at[b], gsem[b])

    def wait_gather(b):
      pltpu.make_async_copy(table_hbm.at[idx_v.at[0]], rows_v.at[b],
                            gsem[b]).wait()

    erows = [lane, lane + 16]

    def transpose(b):
      @plsc.parallel_loop(0, LANES, unroll=2)
      def _(j):
        cj = jnp.broadcast_to(j, (16,))
        for g in range(2):
          v16 = rows_v[b, j, pl.ds(g * 16, 16)]
          plsc.store_scatter(trv.at[b], [erows[g], cj], v16)

    def fire_stores(g, b):
      h = g // TC_PER_H
      tc = g % TC_PER_H
      for tr in range(ETILES):
        pltpu.async_copy(trv.at[b, pl.ds(tr * 8, 8), pl.ds(0, LANES)],
                         out_hbm.at[h, tr, tc], ssem[b])

    def wait_stores(g, b):
      h = g // TC_PER_H
      tc = g % TC_PER_H
      for tr in range(ETILES):
        pltpu.make_async_copy(trv.at[b, pl.ds(tr * 8, 8), pl.ds(0, LANES)],
                              out_hbm.at[h, tr, tc], ssem[b]).wait()

    fire_gather(0, 0)
    fire_gather(1, 1)

    for bsel in range(2):
      wait_gather(bsel)
      transpose(bsel)
      fire_gather(2 + bsel, bsel)
      fire_stores(base + bsel, bsel)

    @pl.loop(1, BLOCKS_PER_W // 2 - 1)
    def _(i):
      for bsel in range(2):
        gl = i * 2 + bsel
        g = base + gl
        wait_gather(bsel)
        wait_stores(g - 2, bsel)
        transpose(bsel)
        fire_gather(gl + 2, bsel)
        fire_stores(g, bsel)

    for bsel in range(2):
      g = base + BLOCKS_PER_W - 2 + bsel
      wait_gather(bsel)
      wait_stores(g - 2, bsel)
      transpose(bsel)
      fire_stores(g, bsel)

    wait_stores(base + BLOCKS_PER_W - 2, 0)
    wait_stores(base + BLOCKS_PER_W - 1, 1)

  return sc_embed


def kernel(x, table):
  idx = x.T.reshape(NBLOCKS, LANES).astype(jnp.int32)
  tail = lax.slice(table, (VFULL, 0), (WORD_NUM, EMBED_DIM)).reshape(-1)
  lin = _make_linearizer()(table.T, tail).reshape(WORD_NUM, EMBED_DIM)
  out5 = _make_sc_kernel()(idx, lin)
  return out5.transpose(2, 4, 0, 1, 3).reshape(BATCH, HIST, EMBED_DIM)


# speedup vs baseline: 1.8044x; 1.0682x over previous
import jax
import jax.numpy as jnp
from jax import lax
from jax.experimental import pallas as pl
from jax.experimental.pallas import tpu as pltpu
from jax.experimental.pallas import tpu_sc as plsc
import functools

BATCH = 16384
HIST = 50
EMBED_DIM = 32

NC = 2
NS = 16
NW = NC * NS

WORD_NUM = 1000000
WORDS = WORD_NUM * EMBED_DIM

LANES = 128
NBLOCKS = HIST * (BATCH // LANES)
BLOCKS_PER_W = NBLOCKS // NW
TC_PER_H = BATCH // LANES
ETILES = EMBED_DIM // 8


VFULL = 999936
VCHUNK = 512
NCH = VFULL // VCHUNK


def _make_linearizer():
  mesh = plsc.VectorSubcoreMesh(
      core_axis_name="c", subcore_axis_name="s",
      num_cores=NC, num_subcores=NS)

  @functools.partial(
      pl.kernel,
      out_type=jax.ShapeDtypeStruct((WORDS,), jnp.float32),
      mesh=mesh,
      scratch_types=[
          pltpu.VMEM((128, 129), jnp.float32),
          pltpu.VMEM((128, 129), jnp.float32),
          pltpu.VMEM((VCHUNK * EMBED_DIM,), jnp.float32),
          pltpu.VMEM((VCHUNK * EMBED_DIM,), jnp.float32),
          pltpu.VMEM(((WORD_NUM - VFULL) * EMBED_DIM,), jnp.float32),
          pltpu.SemaphoreType.DMA,
          pltpu.SemaphoreType.DMA,
          pltpu.SemaphoreType.DMA,
          pltpu.SemaphoreType.DMA,
      ],
      compiler_params=pltpu.CompilerParams(use_tc_tiling_on_sc=True,
                                           needs_layout_passes=False,
                                           disable_bounds_checks=True),
  )
  def sc_lin(tab_t, tail_hbm, out_hbm, inb0, inb1, outb0, outb1, tailv,
             i0, i1, o0, o1):
    inb = [inb0, inb1]
    outb = [outb0, outb1]
    isem = [i0, i1]
    osem = [o0, o1]
    wid = lax.axis_index("s") * NC + lax.axis_index("c")

    def fire_in(k, b):
      v0 = (wid + k * NW) * VCHUNK
      for tr in range(4):
        for tcq in range(4):
          pltpu.async_copy(
              tab_t.at[pl.ds(tr * 8, 8), pl.ds(v0 + tcq * 128, 128)],
              inb[b].at[pl.ds((tr * 4 + tcq) * 8, 8), pl.ds(0, 128)], isem[b])

    def wait_in(k, b):
      v0 = (wid + k * NW) * VCHUNK
      for tr in range(4):
        for tcq in range(4):
          pltpu.make_async_copy(
              tab_t.at[pl.ds(tr * 8, 8), pl.ds(v0 + tcq * 128, 128)],
              inb[b].at[pl.ds((tr * 4 + tcq) * 8, 8), pl.ds(0, 128)], isem[b]).wait()

    lane = lax.iota(jnp.int32, 16)
    erow = [(((g * 16 + lane) // 8) * 32 + (g * 16 + lane) % 8)
            for g in range(2)]

    def transpose(b):
      @plsc.parallel_loop(0, VCHUNK, unroll=2)
      def _(v):
        vcol = jnp.broadcast_to(v % 128, (16,))
        rbase = (v // 128) * 8
        for g in range(2):
          val = plsc.load_gather(inb[b], [erow[g] + rbase, vcol])
          outb[b][pl.ds(v * EMBED_DIM + g * 16, 16)] = val

    def fire_out(k, b):
      v0 = (wid + k * NW) * VCHUNK
      pltpu.async_copy(
          outb[b],
          out_hbm.at[pl.ds(v0 * EMBED_DIM, VCHUNK * EMBED_DIM)], osem[b])

    def wait_out(k, b):
      v0 = (wid + k * NW) * VCHUNK
      pltpu.make_async_copy(
          outb[b],
          out_hbm.at[pl.ds(v0 * EMBED_DIM, VCHUNK * EMBED_DIM)], osem[b]).wait()

    fire_in(0, 0)

    for b in range(2):
      wait_in(b, b)
      fire_in(b + 1, 1 - b)
      transpose(b)
      fire_out(b, b)

    @pl.loop(1, 30)
    def _(p):
      for b in range(2):
        k = p * 2 + b
        wait_in(k, b)
        fire_in(k + 1, 1 - b)
        wait_out(k - 2, b)
        transpose(b)
        fire_out(k, b)

    wait_in(60, 0)
    wait_out(58, 0)
    transpose(0)
    fire_out(60, 0)

    wait_out(59, 1)

    @pl.when(wid == 0)
    def _():
      fire_in(61, 1)
      wait_in(61, 1)
      transpose(1)
      fire_out(61, 1)
      wait_out(61, 1)

    wait_out(60, 0)

    @pl.when(wid == 1)
    def _():
      pltpu.sync_copy(tail_hbm, tailv)
      pltpu.sync_copy(tailv,
                      out_hbm.at[pl.ds(VFULL * EMBED_DIM,
                                       (WORD_NUM - VFULL) * EMBED_DIM)])

  return sc_lin


def _make_sc_kernel():
  mesh = plsc.VectorSubcoreMesh(
      core_axis_name="c", subcore_axis_name="s",
      num_cores=NC, num_subcores=NS)

  @functools.partial(
      pl.kernel,
      out_type=jax.ShapeDtypeStruct((HIST, ETILES, TC_PER_H, 8, LANES),
                                    jnp.float32),
      mesh=mesh,
      scratch_types=[
          pltpu.VMEM((BLOCKS_PER_W, LANES), jnp.int32),
          pltpu.VMEM((2, LANES, EMBED_DIM), jnp.float32),
          pltpu.VMEM((2, EMBED_DIM, 129), jnp.float32),
          pltpu.SemaphoreType.DMA,
          pltpu.SemaphoreType.DMA,
          pltpu.SemaphoreType.DMA,
          pltpu.SemaphoreType.DMA,
      ],
      compiler_params=pltpu.CompilerParams(use_tc_tiling_on_sc=False,
                                           needs_layout_passes=False,
                                           disable_bounds_checks=True),
  )
  def sc_embed(idx_hbm, table_hbm, out_hbm, idx_v, rows_v, trv, g0, g1, s0, s1):
    gsem = [g0, g1]
    ssem = [s0, s1]
    wid = lax.axis_index("s") * NC + lax.axis_index("c")
    base = wid * BLOCKS_PER_W

    pltpu.sync_copy(idx_hbm.at[pl.ds(base, BLOCKS_PER_W)], idx_v)

    lane = lax.iota(jnp.int32, 16)

    def fire_gather(g, b):
      pltpu.async_copy(table_hbm.at[idx_v.at[g]], rows_v.at[b], gsem[b])

    def wait_gather(b):
      pltpu.make_async_copy(table_hbm.at[idx_v.at[0]], rows_v.at[b],
                            gsem[b]).wait()

    erows = [lane, lane + 16]

    def transpose(b):
      @plsc.parallel_loop(0, LANES, unroll=2)
      def _(j):
        cj = jnp.broadcast_to(j, (16,))
        for g in range(2):
          v16 = rows_v[b, j, pl.ds(g * 16, 16)]
          plsc.store_scatter(trv.at[b], [erows[g], cj], v16)

    def fire_stores(g, b):
      h = g // TC_PER_H
      tc = g % TC_PER_H
      for tr in range(ETILES):
        pltpu.async_copy(trv.at[b, pl.ds(tr * 8, 8), pl.ds(0, LANES)],
                         out_hbm.at[h, tr, tc], ssem[b])

    def wait_stores(g, b):
      h = g // TC_PER_H
      tc = g % TC_PER_H
      for tr in range(ETILES):
        pltpu.make_async_copy(trv.at[b, pl.ds(tr * 8, 8), pl.ds(0, LANES)],
                              out_hbm.at[h, tr, tc], ssem[b]).wait()

    fire_gather(0, 0)
    fire_gather(1, 1)

    for bsel in range(2):
      wait_gather(bsel)
      transpose(bsel)
      fire_gather(2 + bsel, bsel)
      fire_stores(base + bsel, bsel)

    @pl.loop(1, BLOCKS_PER_W // 2 - 1)
    def _(i):
      for bsel in range(2):
        gl = i * 2 + bsel
        g = base + gl
        wait_gather(bsel)
        wait_stores(g - 2, bsel)
        transpose(bsel)
        fire_gather(gl + 2, bsel)
        fire_stores(g, bsel)

    for bsel in range(2):
      g = base + BLOCKS_PER_W - 2 + bsel
      wait_gather(bsel)
      wait_stores(g - 2, bsel)
      transpose(bsel)
      fire_stores(g, bsel)

    wait_stores(base + BLOCKS_PER_W - 2, 0)
    wait_stores(base + BLOCKS_PER_W - 1, 1)

  return sc_embed


def kernel(x, table):
  idx = x.T.reshape(NBLOCKS, LANES).astype(jnp.int32)
  tail = lax.slice(table, (VFULL, 0), (WORD_NUM, EMBED_DIM)).reshape(-1)
  lin = _make_linearizer()(table.T, tail).reshape(WORD_NUM, EMBED_DIM)
  out5 = _make_sc_kernel()(idx, lin)
  return out5.transpose(2, 4, 0, 1, 3).reshape(BATCH, HIST, EMBED_DIM)


# speedup vs baseline: 1.8193x; 1.0083x over previous
import jax
import jax.numpy as jnp
from jax import lax
from jax.experimental import pallas as pl
from jax.experimental.pallas import tpu as pltpu
from jax.experimental.pallas import tpu_sc as plsc
import functools

BATCH = 16384
HIST = 50
EMBED_DIM = 32

NC = 2
NS = 16
NW = NC * NS

WORD_NUM = 1000000
WORDS = WORD_NUM * EMBED_DIM

LANES = 128
NBLOCKS = HIST * (BATCH // LANES)
BLOCKS_PER_W = NBLOCKS // NW
TC_PER_H = BATCH // LANES
ETILES = EMBED_DIM // 8


VFULL = 999936
VCHUNK = 512
NCH = VFULL // VCHUNK


def _make_linearizer():
  mesh = plsc.VectorSubcoreMesh(
      core_axis_name="c", subcore_axis_name="s",
      num_cores=NC, num_subcores=NS)

  @functools.partial(
      pl.kernel,
      out_type=jax.ShapeDtypeStruct((WORDS,), jnp.float32),
      mesh=mesh,
      scratch_types=[
          pltpu.VMEM((32, 517), jnp.float32),
          pltpu.VMEM((32, 517), jnp.float32),
          pltpu.VMEM((VCHUNK * EMBED_DIM,), jnp.float32),
          pltpu.VMEM((VCHUNK * EMBED_DIM,), jnp.float32),
          pltpu.VMEM(((WORD_NUM - VFULL) * EMBED_DIM,), jnp.float32),
          pltpu.SemaphoreType.DMA,
          pltpu.SemaphoreType.DMA,
          pltpu.SemaphoreType.DMA,
          pltpu.SemaphoreType.DMA,
      ],
      compiler_params=pltpu.CompilerParams(use_tc_tiling_on_sc=True,
                                           needs_layout_passes=False,
                                           disable_bounds_checks=True),
  )
  def sc_lin(tab_t, tail_hbm, out_hbm, inb0, inb1, outb0, outb1, tailv,
             i0, i1, o0, o1):
    inb = [inb0, inb1]
    outb = [outb0, outb1]
    isem = [i0, i1]
    osem = [o0, o1]
    wid = lax.axis_index("s") * NC + lax.axis_index("c")

    def fire_in(k, b):
      v0 = (wid + k * NW) * VCHUNK
      for tr in range(4):
        pltpu.async_copy(
            tab_t.at[pl.ds(tr * 8, 8), pl.ds(v0, VCHUNK)],
            inb[b].at[pl.ds(tr * 8, 8), pl.ds(0, VCHUNK)], isem[b])

    def wait_in(k, b):
      v0 = (wid + k * NW) * VCHUNK
      for tr in range(4):
        pltpu.make_async_copy(
            tab_t.at[pl.ds(tr * 8, 8), pl.ds(v0, VCHUNK)],
            inb[b].at[pl.ds(tr * 8, 8), pl.ds(0, VCHUNK)], isem[b]).wait()

    lane = lax.iota(jnp.int32, 16)
    erow = [lane + g * 16 for g in range(2)]

    def transpose(b):
      @plsc.parallel_loop(0, VCHUNK, unroll=2)
      def _(v):
        vcol = jnp.broadcast_to(v, (16,))
        for g in range(2):
          val = plsc.load_gather(inb[b], [erow[g], vcol])
          outb[b][pl.ds(v * EMBED_DIM + g * 16, 16)] = val

    def fire_out(k, b):
      v0 = (wid + k * NW) * VCHUNK
      pltpu.async_copy(
          outb[b],
          out_hbm.at[pl.ds(v0 * EMBED_DIM, VCHUNK * EMBED_DIM)], osem[b])

    def wait_out(k, b):
      v0 = (wid + k * NW) * VCHUNK
      pltpu.make_async_copy(
          outb[b],
          out_hbm.at[pl.ds(v0 * EMBED_DIM, VCHUNK * EMBED_DIM)], osem[b]).wait()

    fire_in(0, 0)

    for b in range(2):
      wait_in(b, b)
      fire_in(b + 1, 1 - b)
      transpose(b)
      fire_out(b, b)

    @pl.loop(1, 30)
    def _(p):
      for b in range(2):
        k = p * 2 + b
        wait_in(k, b)
        fire_in(k + 1, 1 - b)
        wait_out(k - 2, b)
        transpose(b)
        fire_out(k, b)

    wait_in(60, 0)
    wait_out(58, 0)
    transpose(0)
    fire_out(60, 0)

    wait_out(59, 1)

    @pl.when(wid == 0)
    def _():
      fire_in(61, 1)
      wait_in(61, 1)
      transpose(1)
      fire_out(61, 1)
      wait_out(61, 1)

    wait_out(60, 0)

    @pl.when(wid == 1)
    def _():
      pltpu.sync_copy(tail_hbm, tailv)
      pltpu.sync_copy(tailv,
                      out_hbm.at[pl.ds(VFULL * EMBED_DIM,
                                       (WORD_NUM - VFULL) * EMBED_DIM)])

  return sc_lin


def _make_sc_kernel():
  mesh = plsc.VectorSubcoreMesh(
      core_axis_name="c", subcore_axis_name="s",
      num_cores=NC, num_subcores=NS)

  @functools.partial(
      pl.kernel,
      out_type=jax.ShapeDtypeStruct((HIST, ETILES, TC_PER_H, 8, LANES),
                                    jnp.float32),
      mesh=mesh,
      scratch_types=[
          pltpu.VMEM((BLOCKS_PER_W, LANES), jnp.int32),
          pltpu.VMEM((2, LANES, EMBED_DIM), jnp.float32),
          pltpu.VMEM((2, EMBED_DIM, 129), jnp.float32),
          pltpu.SemaphoreType.DMA,
          pltpu.SemaphoreType.DMA,
          pltpu.SemaphoreType.DMA,
          pltpu.SemaphoreType.DMA,
      ],
      compiler_params=pltpu.CompilerParams(use_tc_tiling_on_sc=False,
                                           needs_layout_passes=False,
                                           disable_bounds_checks=True),
  )
  def sc_embed(idx_hbm, table_hbm, out_hbm, idx_v, rows_v, trv, g0, g1, s0, s1):
    gsem = [g0, g1]
    ssem = [s0, s1]
    wid = lax.axis_index("s") * NC + lax.axis_index("c")
    base = wid * BLOCKS_PER_W

    pltpu.sync_copy(idx_hbm.at[pl.ds(base, BLOCKS_PER_W)], idx_v)

    lane = lax.iota(jnp.int32, 16)

    def fire_gather(g, b):
      pltpu.async_copy(table_hbm.at[idx_v.at[g]], rows_v.at[b], gsem[b])

    def wait_gather(b):
      pltpu.make_async_copy(table_hbm.at[idx_v.at[0]], rows_v.at[b],
                            gsem[b]).wait()

    erows = [lane, lane + 16]

    def transpose(b):
      @plsc.parallel_loop(0, LANES, unroll=2)
      def _(j):
        cj = jnp.broadcast_to(j, (16,))
        for g in range(2):
          v16 = rows_v[b, j, pl.ds(g * 16, 16)]
          plsc.store_scatter(trv.at[b], [erows[g], cj], v16)

    def fire_stores(g, b):
      h = g // TC_PER_H
      tc = g % TC_PER_H
      for tr in range(ETILES):
        pltpu.async_copy(trv.at[b, pl.ds(tr * 8, 8), pl.ds(0, LANES)],
                         out_hbm.at[h, tr, tc], ssem[b])

    def wait_stores(g, b):
      h = g // TC_PER_H
      tc = g % TC_PER_H
      for tr in range(ETILES):
        pltpu.make_async_copy(trv.at[b, pl.ds(tr * 8, 8), pl.ds(0, LANES)],
                              out_hbm.at[h, tr, tc], ssem[b]).wait()

    fire_gather(0, 0)
    fire_gather(1, 1)

    for bsel in range(2):
      wait_gather(bsel)
      transpose(bsel)
      fire_gather(2 + bsel, bsel)
      fire_stores(base + bsel, bsel)

    @pl.loop(1, BLOCKS_PER_W // 2 - 1)
    def _(i):
      for bsel in range(2):
        gl = i * 2 + bsel
        g = base + gl
        wait_gather(bsel)
        wait_stores(g - 2, bsel)
        transpose(bsel)
        fire_gather(gl + 2, bsel)
        fire_stores(g, bsel)

    for bsel in range(2):
      g = base + BLOCKS_PER_W - 2 + bsel
      wait_gather(bsel)
      wait_stores(g - 2, bsel)
      transpose(bsel)
      fire_stores(g, bsel)

    wait_stores(base + BLOCKS_PER_W - 2, 0)
    wait_stores(base + BLOCKS_PER_W - 1, 1)

  return sc_embed


def kernel(x, table):
  idx = x.T.reshape(NBLOCKS, LANES).astype(jnp.int32)
  tail = lax.slice(table, (VFULL, 0), (WORD_NUM, EMBED_DIM)).reshape(-1)
  lin = _make_linearizer()(table.T, tail).reshape(WORD_NUM, EMBED_DIM)
  out5 = _make_sc_kernel()(idx, lin)
  return out5.transpose(2, 4, 0, 1, 3).reshape(BATCH, HIST, EMBED_DIM)
